# baseline (speedup 1.0000x reference)
import jax
import jax.numpy as jnp
from jax.experimental import pallas as pl
from jax.experimental.pallas import tpu as pltpu
from jax.experimental.pallas import tpu_sc as plsc

W = 128
K = 2


def kernel(visit_segments, table):
    batch, hist = visit_segments.shape
    vocab, embed = table.shape
    n = batch * hist
    n_steps = n // (K * W)
    idx = visit_segments.reshape(n_steps, K, W).astype(jnp.int32)

    @pl.kernel(
        out_type=jax.ShapeDtypeStruct((n, embed), table.dtype),
        mesh=plsc.VectorSubcoreMesh(
            core_axis_name="core", subcore_axis_name="subcore"
        ),
    )
    def gather_kernel(table_hbm, i_hbm, o_hbm):
        def body(i_vmem, o_vmem):
            for j in range(K):
                pltpu.sync_copy(
                    table_hbm.at[i_vmem.at[0, j]],
                    o_vmem.at[pl.ds(j * W, W)],
                )

        pltpu.emit_pipeline(
            body,
            grid=(n_steps,),
            in_specs=[
                pl.BlockSpec((1, K, W), index_map=lambda i: (i, 0, 0))
            ],
            out_specs=[
                pl.BlockSpec((K * W, embed), index_map=lambda i: (i, 0))
            ],
            core_axis_name=("core", "subcore"),
            dimension_semantics=(pltpu.PARALLEL,),
        )(i_hbm, o_hbm)

    out = gather_kernel(table, idx)
    return out.reshape(batch, hist, embed)


# speedup vs baseline: 1.0180x; 1.0180x over previous
import jax
from jax import lax
import jax.numpy as jnp
from jax.experimental import pallas as pl
from jax.experimental.pallas import tpu as pltpu
from jax.experimental.pallas import tpu_sc as plsc

NC = 2
NS = 16
NW = NC * NS
W = 128
NBUF = 4
CHUNK = 160


def kernel(visit_segments, table):
    batch, hist = visit_segments.shape
    vocab, embed = table.shape
    n = batch * hist
    n_win = n // (W * NW)
    n_chunks = n_win // CHUNK

    idx = visit_segments.reshape(n_win, NW * W).astype(jnp.int32)

    scratch = [pltpu.VMEM((CHUNK, W), jnp.int32)]
    scratch += [pltpu.VMEM((W, embed), table.dtype) for _ in range(NBUF)]
    scratch += [pltpu.SemaphoreType.DMA for _ in range(2 * NBUF)]

    @pl.kernel(
        out_type=jax.ShapeDtypeStruct((n, embed), table.dtype),
        mesh=plsc.VectorSubcoreMesh(core_axis_name="c", subcore_axis_name="s"),
        scratch_types=scratch,
    )
    def gather_kernel(table_hbm, idx_hbm, out_hbm, idx_v, *bufs_and_sems):
        rows = bufs_and_sems[:NBUF]
        gsem = bufs_and_sems[NBUF:2 * NBUF]
        wsem = bufs_and_sems[2 * NBUF:]
        wid = lax.axis_index("s") * NC + lax.axis_index("c")

        def out_slice(v):
            return out_hbm.at[pl.ds((v * NW + wid) * W, W)]

        def start_gather(j, r):
            pltpu.async_copy(table_hbm.at[idx_v.at[r]], rows[j], gsem[j])

        def wait_gather(j):
            pltpu.make_async_copy(table_hbm.at[idx_v.at[0]], rows[j],
                                  gsem[j]).wait()

        def start_write(j, v):
            pltpu.async_copy(rows[j], out_slice(v), wsem[j])

        def wait_write(j, v):
            pltpu.make_async_copy(rows[j], out_slice(v), wsem[j]).wait()

        @pl.loop(0, n_chunks)
        def _(c):
            c0 = c * CHUNK
            pltpu.sync_copy(
                idx_hbm.at[pl.ds(c0, CHUNK), pl.ds(wid * W, W)], idx_v
            )

            for j in range(NBUF):
                start_gather(j, j)

            @pl.loop(NBUF, CHUNK, step=NBUF)
            def _(v):
                for j in range(NBUF):
                    wait_gather(j)
                    start_write(j, c0 + v - NBUF + j)
                for j in range(NBUF):
                    wait_write(j, c0 + v - NBUF + j)
                    start_gather(j, v + j)

            for j in range(NBUF):
                wait_gather(j)
                start_write(j, c0 + CHUNK - NBUF + j)
            for j in range(NBUF):
                wait_write(j, c0 + CHUNK - NBUF + j)

    out = gather_kernel(table, idx)
    return out.reshape(batch, hist, embed)


# speedup vs baseline: 1.2446x; 1.2226x over previous
import jax
import jax.numpy as jnp
from jax.experimental import pallas as pl
from jax.experimental.pallas import tpu as pltpu
from jax.experimental.pallas import tpu_sc as plsc

WINDOW = 128


def kernel(visit_segments, table):
    batch, hist = visit_segments.shape
    vocab, embed = table.shape
    n = batch * hist
    idx = visit_segments.reshape(1, n).astype(jnp.int32)

    @pl.kernel(
        out_type=jax.ShapeDtypeStruct((n, embed), table.dtype),
        mesh=plsc.VectorSubcoreMesh(
            core_axis_name="core", subcore_axis_name="subcore"
        ),
    )
    def gather_kernel(table_hbm, i_hbm, o_hbm):
        def body(i_vmem, o_vmem):
            pltpu.sync_copy(table_hbm.at[i_vmem.at[0]], o_vmem)

        pltpu.emit_pipeline(
            body,
            grid=(n // WINDOW,),
            in_specs=[pl.BlockSpec((1, WINDOW), index_map=lambda i: (0, i))],
            out_specs=[pl.BlockSpec((WINDOW, embed), index_map=lambda i: (i, 0))],
            core_axis_name=("core", "subcore"),
            dimension_semantics=(pltpu.PARALLEL,),
        )(i_hbm, o_hbm)

    out = gather_kernel(table, idx)
    return out.reshape(batch, hist, embed)
